# baseline (speedup 1.0000x reference)
import jax
import jax.numpy as jnp
from jax import lax
from jax.experimental import pallas as pl
from jax.experimental.pallas import tpu as pltpu
from jax.experimental.pallas import tpu_sc as plsc

N_NODES_C = 10000
N_EDGES_C = 320000
D_C = 128
DH = D_C // 2

NC = 2
NS = 16
EPT = N_EDGES_C // NS
CHUNK = 256
NFULL = EPT // CHUNK
NPAIR = NFULL // 2
NPAIR_C0 = 20
TAIL = EPT - NFULL * CHUNK
N_PAD = 10000
ROWS_PER_TILE = N_PAD // NS
WB = 125
DEG_W = 16


def _sc_body(featv_hbm, src_hbm, dst_hbm, zsum_hbm, zdeg_hbm, ones_hbm,
             out_sum, out_deg,
             src_idx0, dst_idx0, rows0, src_idx1, dst_idx1, rows1,
             src_idx_t, dst_idx_t, rows_t, ones_v, zdeg_v,
             acc_sum, acc_deg,
             sem_g0, sem_g1, sem_s0, sem_s1, sem_d):
    c = lax.axis_index("c")
    s = lax.axis_index("s")

    pltpu.sync_copy(zsum_hbm, rows0.at[pl.ds(0, WB)])
    pltpu.sync_copy(zdeg_hbm, zdeg_v)
    pltpu.sync_copy(ones_hbm, ones_v)

    @pl.loop(0, ROWS_PER_TILE // WB)
    def _(j):
        r0 = s * ROWS_PER_TILE + j * WB
        pltpu.sync_copy(rows0.at[pl.ds(0, WB)], acc_sum.at[pl.ds(r0, WB)])
        pltpu.sync_copy(zdeg_v, acc_deg.at[pl.ds(r0, WB)])

    plsc.subcore_barrier()

    base0 = s * EPT

    @pl.loop(0, NPAIR)
    def _(j):
        ba = base0 + 2 * j * CHUNK
        bb = ba + CHUNK
        pltpu.sync_copy(src_hbm.at[pl.ds(ba, CHUNK)], src_idx0)
        pltpu.sync_copy(dst_hbm.at[pl.ds(ba, CHUNK)], dst_idx0)
        pltpu.sync_copy(src_hbm.at[pl.ds(bb, CHUNK)], src_idx1)
        pltpu.sync_copy(dst_hbm.at[pl.ds(bb, CHUNK)], dst_idx1)

        @pl.loop(0, CHUNK // 16)
        def _(k):
            v = src_idx0[pl.ds(k * 16, 16)]
            src_idx0[pl.ds(k * 16, 16)] = v + v + c
            w = src_idx1[pl.ds(k * 16, 16)]
            src_idx1[pl.ds(k * 16, 16)] = w + w + c

        g0 = pltpu.async_copy(featv_hbm.at[src_idx0], rows0, sem_g0)
        g1 = pltpu.async_copy(featv_hbm.at[src_idx1], rows1, sem_g1)

        @pl.when(jnp.logical_or(
            jnp.logical_and(c == 0, j < NPAIR_C0),
            jnp.logical_and(c == 1, j >= NPAIR_C0)))
        def _():
            pltpu.async_copy(ones_v, acc_deg.at[dst_idx0], sem_d,
                             add=True).wait()
            pltpu.async_copy(ones_v, acc_deg.at[dst_idx1], sem_d,
                             add=True).wait()

        g0.wait()
        s0 = pltpu.async_copy(rows0, acc_sum.at[dst_idx0], sem_s0, add=True)
        g1.wait()
        s1 = pltpu.async_copy(rows1, acc_sum.at[dst_idx1], sem_s1, add=True)
        s0.wait()
        s1.wait()

    tb = base0 + NFULL * CHUNK
    pltpu.sync_copy(src_hbm.at[pl.ds(tb, TAIL)], src_idx_t)
    pltpu.sync_copy(dst_hbm.at[pl.ds(tb, TAIL)], dst_idx_t)

    @pl.loop(0, TAIL // 16)
    def _(k):
        v = src_idx_t[pl.ds(k * 16, 16)]
        src_idx_t[pl.ds(k * 16, 16)] = v + v + c

    pltpu.async_copy(featv_hbm.at[src_idx_t], rows_t, sem_g0).wait()
    pltpu.sync_copy(rows_t, acc_sum.at[dst_idx_t], add=True)

    @pl.when(c == 1)
    def _():
        pltpu.sync_copy(ones_v.at[pl.ds(0, TAIL)], acc_deg.at[dst_idx_t],
                        add=True)

    plsc.subcore_barrier()

    @pl.loop(0, ROWS_PER_TILE // WB)
    def _(j):
        r0 = s * ROWS_PER_TILE + j * WB
        pltpu.sync_copy(acc_sum.at[pl.ds(r0, WB)], rows0.at[pl.ds(0, WB)])
        pltpu.sync_copy(rows0.at[pl.ds(0, WB)], out_sum.at[c, pl.ds(r0, WB)])
        pltpu.sync_copy(acc_deg.at[pl.ds(r0, WB)], zdeg_v)
        pltpu.sync_copy(zdeg_v, out_deg.at[c, pl.ds(r0, WB)])


@jax.jit
def _sc_aggregate(featv, src, dst, zsum, zdeg, ones):
    mesh = plsc.VectorSubcoreMesh(core_axis_name="c", subcore_axis_name="s")
    k = pl.kernel(
        _sc_body,
        out_type=(
            jax.ShapeDtypeStruct((NC, N_PAD, DH), jnp.float32),
            jax.ShapeDtypeStruct((NC, N_PAD, DEG_W), jnp.float32),
        ),
        mesh=mesh,
        scratch_types=[
            pltpu.VMEM((CHUNK,), jnp.int32),
            pltpu.VMEM((CHUNK,), jnp.int32),
            pltpu.VMEM((CHUNK, DH), jnp.float32),
            pltpu.VMEM((CHUNK,), jnp.int32),
            pltpu.VMEM((CHUNK,), jnp.int32),
            pltpu.VMEM((CHUNK, DH), jnp.float32),
            pltpu.VMEM((TAIL,), jnp.int32),
            pltpu.VMEM((TAIL,), jnp.int32),
            pltpu.VMEM((TAIL, DH), jnp.float32),
            pltpu.VMEM((CHUNK, DEG_W), jnp.float32),
            pltpu.VMEM((WB, DEG_W), jnp.float32),
            pltpu.VMEM_SHARED((N_PAD, DH), jnp.float32),
            pltpu.VMEM_SHARED((N_PAD, DEG_W), jnp.float32),
            pltpu.SemaphoreType.DMA,
            pltpu.SemaphoreType.DMA,
            pltpu.SemaphoreType.DMA,
            pltpu.SemaphoreType.DMA,
            pltpu.SemaphoreType.DMA,
        ],
        compiler_params=pltpu.CompilerParams(use_tc_tiling_on_sc=False),
    )
    return k(featv, src, dst, zsum, zdeg, ones)


def _tc_self_body(feat_ref, ws_ref, b_ref, out_ref):
    out_ref[...] = (
        jnp.dot(feat_ref[...], ws_ref[...], preferred_element_type=jnp.float32)
        + b_ref[...]
    )


def _tc_final_body(selfp_ref, sum_ref, deg_ref, wn_ref, out_ref):
    ssum = jnp.concatenate([sum_ref[0], sum_ref[1]], axis=1)
    deg = deg_ref[0][:, 0:1] + deg_ref[1][:, 0:1]
    h = ssum / jnp.maximum(deg, 1.0)
    out_ref[...] = selfp_ref[...] + jnp.dot(
        h, wn_ref[...], preferred_element_type=jnp.float32)


@jax.jit
def _tc_self(feat, W_self, b2d):
    rb = 2000
    grid = (N_NODES_C // rb,)
    return pl.pallas_call(
        _tc_self_body,
        grid=grid,
        in_specs=[
            pl.BlockSpec((rb, D_C), lambda i: (i, 0)),
            pl.BlockSpec((D_C, D_C), lambda i: (0, 0)),
            pl.BlockSpec((1, D_C), lambda i: (0, 0)),
        ],
        out_specs=pl.BlockSpec((rb, D_C), lambda i: (i, 0)),
        out_shape=jax.ShapeDtypeStruct((N_NODES_C, D_C), jnp.float32),
    )(feat, W_self, b2d)


@jax.jit
def _tc_final(selfp, part_sum, part_deg, W_neigh):
    rb = 2000
    grid = (N_NODES_C // rb,)
    return pl.pallas_call(
        _tc_final_body,
        grid=grid,
        in_specs=[
            pl.BlockSpec((rb, D_C), lambda i: (i, 0)),
            pl.BlockSpec((NC, rb, DH), lambda i: (0, i, 0)),
            pl.BlockSpec((NC, rb, DEG_W), lambda i: (0, i, 0)),
            pl.BlockSpec((D_C, D_C), lambda i: (0, 0)),
        ],
        out_specs=pl.BlockSpec((rb, D_C), lambda i: (i, 0)),
        out_shape=jax.ShapeDtypeStruct((N_NODES_C, D_C), jnp.float32),
    )(selfp, part_sum, part_deg, W_neigh)


def kernel(feat, edge_index, W_self, W_neigh, b_neigh):
    ei = edge_index.astype(jnp.int32)
    src = ei[0]
    dst = ei[1]
    featv = feat.reshape(2 * N_NODES_C, DH)
    zsum = jnp.zeros((WB, DH), jnp.float32)
    zdeg = jnp.zeros((WB, DEG_W), jnp.float32)
    ones = jnp.ones((CHUNK, DEG_W), jnp.float32)
    b2d = b_neigh.reshape(1, D_C)
    part_sum, part_deg = _sc_aggregate(featv, src, dst, zsum, zdeg, ones)
    selfp = _tc_self(feat, W_self, b2d)
    return _tc_final(selfp, part_sum, part_deg, W_neigh)
